# Initial kernel scaffold; baseline (speedup 1.0000x reference)
import jax
import jax.numpy as jnp
from jax.experimental import pallas as pl


def kernel(x, edge_index, W1, b1, W2, b2):
    raise NotImplementedError("write your pallas kernel here")


# speedup vs baseline: 48.2061x; 48.2061x over previous
import functools

import jax
import jax.numpy as jnp
from jax import lax
from jax.experimental import pallas as pl
from jax.experimental.pallas import tpu as pltpu
from jax.experimental.pallas import tpu_sc as plsc

N = 100000
E = 6400000
IN_DIM = 6
HID = 32
EMB = 32

NPAD = 96
NT = N + NPAD
NCORE = 2
NSUB = 16
NWORK = NCORE * NSUB
CH = 2048
J = CH // 128
EPW = 98 * CH
EPAD = EPW * NWORK
ITERS = EPW // CH
RPT = NT // NSUB

_mesh = plsc.VectorSubcoreMesh(core_axis_name="c", subcore_axis_name="s")
_sc_params = pltpu.CompilerParams(use_tc_tiling_on_sc=False)


@functools.partial(
    pl.kernel,
    out_type=jax.ShapeDtypeStruct((NCORE, NT, 8), jnp.float32),
    mesh=_mesh,
    compiler_params=_sc_params,
    scratch_types=[
        pltpu.VMEM_SHARED((NT, 8), jnp.float32),
        pltpu.VMEM((CH,), jnp.int32),
        pltpu.VMEM((J, 128), jnp.int32),
        pltpu.VMEM((CH, 8), jnp.float32),
        pltpu.SemaphoreType.DMA,
    ],
)
def _pass_a(src_hbm, dst2_hbm, xpad_hbm, zeros8_hbm, out_hbm,
            acc, srcb, dstb, rows, sem):
    cid = lax.axis_index("c")
    sid = lax.axis_index("s")
    r0 = sid * RPT

    @pl.when(cid == 0)
    def _():
        pltpu.sync_copy(xpad_hbm.at[pl.ds(r0, RPT)], acc.at[pl.ds(r0, RPT)])

    @pl.when(cid != 0)
    def _():
        pltpu.sync_copy(zeros8_hbm.at[pl.ds(r0, RPT)], acc.at[pl.ds(r0, RPT)])

    plsc.subcore_barrier()

    base = (cid * NSUB + sid) * EPW

    @pl.loop(0, ITERS)
    def _(i):
        off = base + i * CH
        pltpu.sync_copy(src_hbm.at[pl.ds(off, CH)], srcb)
        pltpu.async_copy(xpad_hbm.at[srcb], rows, sem).wait()
        row0 = pl.multiple_of(off // 128, 8)
        pltpu.sync_copy(dst2_hbm.at[pl.ds(row0, J)], dstb)
        for j in range(J):
            pltpu.sync_copy(rows.at[pl.ds(j * 128, 128)],
                            acc.at[dstb.at[j]], add=True)

    plsc.subcore_barrier()
    pltpu.sync_copy(acc.at[pl.ds(r0, RPT)], out_hbm.at[cid, pl.ds(r0, RPT)])


@functools.partial(
    pl.kernel,
    out_type=jax.ShapeDtypeStruct((NCORE, NT), jnp.float32),
    mesh=_mesh,
    compiler_params=_sc_params,
    scratch_types=[
        pltpu.VMEM_SHARED((NT,), jnp.float32),
        pltpu.VMEM((CH,), jnp.int32),
        pltpu.VMEM((J, 128), jnp.int32),
        pltpu.VMEM((CH,), jnp.float32),
        pltpu.SemaphoreType.DMA,
    ],
)
def _pass_b(dst_hbm, src2_hbm, invt_hbm, zeros1_hbm, out_hbm,
            cacc, dstb, srcb, vals, sem):
    cid = lax.axis_index("c")
    sid = lax.axis_index("s")
    r0 = sid * RPT
    pltpu.sync_copy(zeros1_hbm.at[pl.ds(r0, RPT)], cacc.at[pl.ds(r0, RPT)])
    plsc.subcore_barrier()

    base = (cid * NSUB + sid) * EPW

    @pl.loop(0, ITERS)
    def _(i):
        off = base + i * CH
        pltpu.sync_copy(dst_hbm.at[pl.ds(off, CH)], dstb)
        pltpu.async_copy(invt_hbm.at[dstb], vals, sem).wait()
        row0 = pl.multiple_of(off // 128, 8)
        pltpu.sync_copy(src2_hbm.at[pl.ds(row0, J)], srcb)
        for j in range(J):
            pltpu.sync_copy(vals.at[pl.ds(j * 128, 128)],
                            cacc.at[srcb.at[j]], add=True)

    plsc.subcore_barrier()
    pltpu.sync_copy(cacc.at[pl.ds(r0, RPT)], out_hbm.at[cid, pl.ds(r0, RPT)])


BLK = 800


def _inv_body(spart_ref, inv_ref):
    s6 = spart_ref[0, :, 6:7] + spart_ref[1, :, 6:7]
    inv_ref[...] = 1.0 / s6


def _final_body(spart_ref, inv_ref, cpart_ref, w1_ref, b1_ref, w2_ref,
                b2_ref, out_ref, acc):
    i = pl.program_id(0)
    s = spart_ref[0] + spart_ref[1]
    inv = inv_ref[...]
    c = cpart_ref[0, :, 0] + cpart_ref[1, :, 0]
    g = (c + inv[:, 0]) * (1.0 / N)
    rst = s * inv
    z = jnp.dot(rst, w1_ref[...], preferred_element_type=jnp.float32)
    h1 = jnp.maximum(z + b1_ref[...], 0.0)
    part = jnp.sum(h1 * g[:, None], axis=0, keepdims=True)

    @pl.when(i == 0)
    def _():
        acc[...] = jnp.zeros_like(acc)

    acc[...] += part

    @pl.when(i == pl.num_programs(0) - 1)
    def _():
        out_ref[...] = (
            jnp.dot(acc[...], w2_ref[...], preferred_element_type=jnp.float32)
            + b2_ref[...]
        )


def kernel(x, edge_index, W1, b1, W2, b2):
    src = edge_index[0].astype(jnp.int32)
    dst = edge_index[1].astype(jnp.int32)
    pad_idx = N + (jnp.arange(EPAD - E, dtype=jnp.int32) % NPAD)
    src_p = jnp.concatenate([src, pad_idx])
    dst_p = jnp.concatenate([dst, pad_idx])
    src2d = src_p.reshape(-1, 128)
    dst2d = dst_p.reshape(-1, 128)

    xpad = jnp.concatenate(
        [x.astype(jnp.float32),
         jnp.ones((N, 1), jnp.float32),
         jnp.zeros((N, 1), jnp.float32)], axis=1)
    xpad = jnp.pad(xpad, ((0, NPAD), (0, 0)))
    zeros8 = jnp.zeros((NT, 8), jnp.float32)
    zeros1 = jnp.zeros((NT,), jnp.float32)

    spart = _pass_a(src_p, dst2d, xpad, zeros8)
    spart = spart[:, :N]

    inv = pl.pallas_call(
        _inv_body,
        grid=(N // BLK,),
        in_specs=[pl.BlockSpec((NCORE, BLK, 8), lambda i: (0, i, 0))],
        out_specs=pl.BlockSpec((BLK, 1), lambda i: (i, 0)),
        out_shape=jax.ShapeDtypeStruct((N, 1), jnp.float32),
    )(spart)

    invt = jnp.pad(inv[:, 0], (0, NPAD), constant_values=1.0)

    cpart = _pass_b(dst_p, src2d, invt, zeros1)
    cpart = cpart[:, :N, None]

    w1p = jnp.pad(W1.astype(jnp.float32), ((0, 2), (0, 0)))

    out = pl.pallas_call(
        _final_body,
        grid=(N // BLK,),
        in_specs=[
            pl.BlockSpec((NCORE, BLK, 8), lambda i: (0, i, 0)),
            pl.BlockSpec((BLK, 1), lambda i: (i, 0)),
            pl.BlockSpec((NCORE, BLK, 1), lambda i: (0, i, 0)),
            pl.BlockSpec((8, HID), lambda i: (0, 0)),
            pl.BlockSpec((1, HID), lambda i: (0, 0)),
            pl.BlockSpec((HID, EMB), lambda i: (0, 0)),
            pl.BlockSpec((1, EMB), lambda i: (0, 0)),
        ],
        out_specs=pl.BlockSpec((1, EMB), lambda i: (0, 0)),
        out_shape=jax.ShapeDtypeStruct((1, EMB), jnp.float32),
        scratch_shapes=[pltpu.VMEM((1, EMB), jnp.float32)],
    )(spart, inv, cpart, w1p, b1.reshape(1, HID), W2.astype(jnp.float32),
      b2.reshape(1, EMB))

    return out


# speedup vs baseline: 61.8458x; 1.2829x over previous
import functools

import jax
import jax.numpy as jnp
from jax import lax
from jax.experimental import pallas as pl
from jax.experimental.pallas import tpu as pltpu
from jax.experimental.pallas import tpu_sc as plsc

N = 100000
E = 6400000
IN_DIM = 6
HID = 32
EMB = 32

NPAD = 96
NT = N + NPAD
NCORE = 2
NSUB = 16
NWORK = NCORE * NSUB
CH = 1024
J = CH // 128
EPW = 196 * CH
EPAD = EPW * NWORK
ITERS = EPW // CH
RPT = NT // NSUB

_mesh = plsc.VectorSubcoreMesh(core_axis_name="c", subcore_axis_name="s")
_sc_params = pltpu.CompilerParams(use_tc_tiling_on_sc=False)


@functools.partial(
    pl.kernel,
    out_type=jax.ShapeDtypeStruct((NCORE, NT, 8), jnp.float32),
    mesh=_mesh,
    compiler_params=_sc_params,
    scratch_types=[
        pltpu.VMEM_SHARED((NT, 8), jnp.float32),
        pltpu.VMEM((2, CH), jnp.int32),
        pltpu.VMEM((2, J, 128), jnp.int32),
        pltpu.VMEM((2, CH, 8), jnp.float32),
        pltpu.SemaphoreType.DMA,
        pltpu.SemaphoreType.DMA,
        pltpu.SemaphoreType.DMA,
    ],
)
def _pass_a(src_hbm, dst2_hbm, xpad_hbm, zeros8_hbm, out_hbm,
            acc, srcb, dstb, rows, sem_i, sem_g, sem_s):
    cid = lax.axis_index("c")
    sid = lax.axis_index("s")
    r0 = sid * RPT

    @pl.when(cid == 0)
    def _():
        pltpu.sync_copy(xpad_hbm.at[pl.ds(r0, RPT)], acc.at[pl.ds(r0, RPT)])

    @pl.when(cid != 0)
    def _():
        pltpu.sync_copy(zeros8_hbm.at[pl.ds(r0, RPT)], acc.at[pl.ds(r0, RPT)])

    plsc.subcore_barrier()

    base = (cid * NSUB + sid) * EPW

    def fire_idx(i, b):
        off = base + i * CH
        pltpu.async_copy(src_hbm.at[pl.ds(off, CH)], srcb.at[b], sem_i)
        row0 = pl.multiple_of(off // 128, 8)
        pltpu.async_copy(dst2_hbm.at[pl.ds(row0, J)], dstb.at[b], sem_i)

    def wait_idx(b):
        pltpu.make_async_copy(src_hbm.at[pl.ds(0, CH)], srcb.at[b],
                              sem_i).wait()
        pltpu.make_async_copy(dst2_hbm.at[pl.ds(0, J)], dstb.at[b],
                              sem_i).wait()

    def fire_gather(b):
        pltpu.async_copy(xpad_hbm.at[srcb.at[b]], rows.at[b], sem_g)

    def wait_gather(b):
        pltpu.make_async_copy(xpad_hbm.at[srcb.at[b]], rows.at[b],
                              sem_g).wait()

    fire_idx(0, 0)
    wait_idx(0)
    fire_gather(0)
    fire_idx(1, 1)

    @pl.loop(0, ITERS // 2)
    def _(k):
        for b in range(2):
            i = 2 * k + b
            wait_gather(b)
            nb = 1 - b

            @pl.when(i + 1 < ITERS)
            def _():
                wait_idx(nb)
                fire_gather(nb)

            for j in range(J):
                pltpu.async_copy(rows.at[b, pl.ds(j * 128, 128)],
                                 acc.at[dstb.at[b, j]], sem_s, add=True)
            for j in range(J):
                pltpu.make_async_copy(rows.at[b, pl.ds(j * 128, 128)],
                                      acc.at[dstb.at[b, j]], sem_s).wait()

            @pl.when(i + 2 < ITERS)
            def _():
                fire_idx(i + 2, b)

    plsc.subcore_barrier()
    pltpu.sync_copy(acc.at[pl.ds(r0, RPT)], out_hbm.at[cid, pl.ds(r0, RPT)])


@functools.partial(
    pl.kernel,
    out_type=jax.ShapeDtypeStruct((NCORE, NT), jnp.float32),
    mesh=_mesh,
    compiler_params=_sc_params,
    scratch_types=[
        pltpu.VMEM_SHARED((NT,), jnp.float32),
        pltpu.VMEM((2, CH), jnp.int32),
        pltpu.VMEM((2, J, 128), jnp.int32),
        pltpu.VMEM((2, CH), jnp.float32),
        pltpu.SemaphoreType.DMA,
        pltpu.SemaphoreType.DMA,
        pltpu.SemaphoreType.DMA,
    ],
)
def _pass_b(dst_hbm, src2_hbm, invt_hbm, zeros1_hbm, out_hbm,
            cacc, dstb, srcb, vals, sem_i, sem_g, sem_s):
    cid = lax.axis_index("c")
    sid = lax.axis_index("s")
    r0 = sid * RPT
    pltpu.sync_copy(zeros1_hbm.at[pl.ds(r0, RPT)], cacc.at[pl.ds(r0, RPT)])
    plsc.subcore_barrier()

    base = (cid * NSUB + sid) * EPW

    def fire_idx(i, b):
        off = base + i * CH
        pltpu.async_copy(dst_hbm.at[pl.ds(off, CH)], dstb.at[b], sem_i)
        row0 = pl.multiple_of(off // 128, 8)
        pltpu.async_copy(src2_hbm.at[pl.ds(row0, J)], srcb.at[b], sem_i)

    def wait_idx(b):
        pltpu.make_async_copy(dst_hbm.at[pl.ds(0, CH)], dstb.at[b],
                              sem_i).wait()
        pltpu.make_async_copy(src2_hbm.at[pl.ds(0, J)], srcb.at[b],
                              sem_i).wait()

    def fire_gather(b):
        pltpu.async_copy(invt_hbm.at[dstb.at[b]], vals.at[b], sem_g)

    def wait_gather(b):
        pltpu.make_async_copy(invt_hbm.at[dstb.at[b]], vals.at[b],
                              sem_g).wait()

    fire_idx(0, 0)
    wait_idx(0)
    fire_gather(0)
    fire_idx(1, 1)

    @pl.loop(0, ITERS // 2)
    def _(k):
        for b in range(2):
            i = 2 * k + b
            wait_gather(b)
            nb = 1 - b

            @pl.when(i + 1 < ITERS)
            def _():
                wait_idx(nb)
                fire_gather(nb)

            for j in range(J):
                pltpu.async_copy(vals.at[b, pl.ds(j * 128, 128)],
                                 cacc.at[srcb.at[b, j]], sem_s, add=True)
            for j in range(J):
                pltpu.make_async_copy(vals.at[b, pl.ds(j * 128, 128)],
                                      cacc.at[srcb.at[b, j]], sem_s).wait()

            @pl.when(i + 2 < ITERS)
            def _():
                fire_idx(i + 2, b)

    plsc.subcore_barrier()
    pltpu.sync_copy(cacc.at[pl.ds(r0, RPT)], out_hbm.at[cid, pl.ds(r0, RPT)])


BLK = 800


def _inv_body(spart_ref, inv_ref):
    s6 = spart_ref[0, :, 6:7] + spart_ref[1, :, 6:7]
    inv_ref[...] = 1.0 / s6


def _final_body(spart_ref, inv_ref, cpart_ref, w1_ref, b1_ref, w2_ref,
                b2_ref, out_ref, acc):
    i = pl.program_id(0)
    s = spart_ref[0] + spart_ref[1]
    inv = inv_ref[...]
    c = cpart_ref[0, :, 0] + cpart_ref[1, :, 0]
    g = (c + inv[:, 0]) * (1.0 / N)
    rst = s * inv
    z = jnp.dot(rst, w1_ref[...], preferred_element_type=jnp.float32)
    h1 = jnp.maximum(z + b1_ref[...], 0.0)
    part = jnp.sum(h1 * g[:, None], axis=0, keepdims=True)

    @pl.when(i == 0)
    def _():
        acc[...] = jnp.zeros_like(acc)

    acc[...] += part

    @pl.when(i == pl.num_programs(0) - 1)
    def _():
        out_ref[...] = (
            jnp.dot(acc[...], w2_ref[...], preferred_element_type=jnp.float32)
            + b2_ref[...]
        )


def kernel(x, edge_index, W1, b1, W2, b2):
    src = edge_index[0].astype(jnp.int32)
    dst = edge_index[1].astype(jnp.int32)
    pad_idx = N + (jnp.arange(EPAD - E, dtype=jnp.int32) % NPAD)
    src_p = jnp.concatenate([src, pad_idx])
    dst_p = jnp.concatenate([dst, pad_idx])
    src2d = src_p.reshape(-1, 128)
    dst2d = dst_p.reshape(-1, 128)

    xpad = jnp.concatenate(
        [x.astype(jnp.float32),
         jnp.ones((N, 1), jnp.float32),
         jnp.zeros((N, 1), jnp.float32)], axis=1)
    xpad = jnp.pad(xpad, ((0, NPAD), (0, 0)))
    zeros8 = jnp.zeros((NT, 8), jnp.float32)
    zeros1 = jnp.zeros((NT,), jnp.float32)

    spart = _pass_a(src_p, dst2d, xpad, zeros8)
    spart = spart[:, :N]

    inv = pl.pallas_call(
        _inv_body,
        grid=(N // BLK,),
        in_specs=[pl.BlockSpec((NCORE, BLK, 8), lambda i: (0, i, 0))],
        out_specs=pl.BlockSpec((BLK, 1), lambda i: (i, 0)),
        out_shape=jax.ShapeDtypeStruct((N, 1), jnp.float32),
    )(spart)

    invt = jnp.pad(inv[:, 0], (0, NPAD), constant_values=1.0)

    cpart = _pass_b(dst_p, src2d, invt, zeros1)
    cpart = cpart[:, :N, None]

    w1p = jnp.pad(W1.astype(jnp.float32), ((0, 2), (0, 0)))

    out = pl.pallas_call(
        _final_body,
        grid=(N // BLK,),
        in_specs=[
            pl.BlockSpec((NCORE, BLK, 8), lambda i: (0, i, 0)),
            pl.BlockSpec((BLK, 1), lambda i: (i, 0)),
            pl.BlockSpec((NCORE, BLK, 1), lambda i: (0, i, 0)),
            pl.BlockSpec((8, HID), lambda i: (0, 0)),
            pl.BlockSpec((1, HID), lambda i: (0, 0)),
            pl.BlockSpec((HID, EMB), lambda i: (0, 0)),
            pl.BlockSpec((1, EMB), lambda i: (0, 0)),
        ],
        out_specs=pl.BlockSpec((1, EMB), lambda i: (0, 0)),
        out_shape=jax.ShapeDtypeStruct((1, EMB), jnp.float32),
        scratch_shapes=[pltpu.VMEM((1, EMB), jnp.float32)],
    )(spart, inv, cpart, w1p, b1.reshape(1, HID), W2.astype(jnp.float32),
      b2.reshape(1, EMB))

    return out


# speedup vs baseline: 85.5515x; 1.3833x over previous
import functools

import jax
import jax.numpy as jnp
from jax import lax
from jax.experimental import pallas as pl
from jax.experimental.pallas import tpu as pltpu
from jax.experimental.pallas import tpu_sc as plsc

N = 100000
E = 6400000
IN_DIM = 6
HID = 32
EMB = 32

NPAD = 96
NT = N + NPAD
NCORE = 2
NSUB = 16
NWORK = NCORE * NSUB
CH = 1024
J = CH // 128
EPW = 196 * CH
ITERS = 196
ITERS_LAST = 174
RPT = NT // NSUB

_mesh = plsc.VectorSubcoreMesh(core_axis_name="c", subcore_axis_name="s")
_sc_params = pltpu.CompilerParams(use_tc_tiling_on_sc=False,
                                  needs_layout_passes=False)


@functools.partial(
    pl.kernel,
    out_type=jax.ShapeDtypeStruct((NCORE, NT, 8), jnp.float32),
    mesh=_mesh,
    compiler_params=_sc_params,
    scratch_types=[
        pltpu.VMEM_SHARED((NT, 8), jnp.float32),
        pltpu.VMEM((2, CH), jnp.int32),
        pltpu.VMEM((2, J, 128), jnp.int32),
        pltpu.VMEM((2, CH, 8), jnp.float32),
        pltpu.SemaphoreType.DMA,
        pltpu.SemaphoreType.DMA,
        pltpu.SemaphoreType.DMA,
    ],
)
def _pass_a(src_hbm, dst2_hbm, xpad_hbm, zeros8_hbm, out_hbm,
            acc, srcb, dstb, rows, sem_i, sem_g, sem_s):
    cid = lax.axis_index("c")
    sid = lax.axis_index("s")
    r0 = sid * RPT

    @pl.when(cid == 0)
    def _():
        pltpu.sync_copy(xpad_hbm.at[pl.ds(r0, RPT)], acc.at[pl.ds(r0, RPT)])

    @pl.when(cid != 0)
    def _():
        pltpu.sync_copy(zeros8_hbm.at[pl.ds(r0, RPT)], acc.at[pl.ds(r0, RPT)])

    plsc.subcore_barrier()

    w = cid * NSUB + sid
    base = w * EPW
    nch = jnp.where(w == NWORK - 1, ITERS_LAST, ITERS)

    def fire_idx(i, b):
        off = base + i * CH
        pltpu.async_copy(src_hbm.at[pl.ds(off, CH)], srcb.at[b], sem_i)
        row0 = pl.multiple_of(off // 128, 8)
        pltpu.async_copy(dst2_hbm.at[pl.ds(row0, J)], dstb.at[b], sem_i)

    def wait_idx(b):
        pltpu.make_async_copy(src_hbm.at[pl.ds(0, CH)], srcb.at[b],
                              sem_i).wait()
        pltpu.make_async_copy(dst2_hbm.at[pl.ds(0, J)], dstb.at[b],
                              sem_i).wait()

    def fire_gather(b):
        pltpu.async_copy(xpad_hbm.at[srcb.at[b]], rows.at[b], sem_g)

    def wait_gather(b):
        pltpu.make_async_copy(xpad_hbm.at[srcb.at[b]], rows.at[b],
                              sem_g).wait()

    fire_idx(0, 0)
    wait_idx(0)
    fire_gather(0)
    fire_idx(1, 1)

    @pl.loop(0, nch // 2)
    def _(k):
        for b in range(2):
            i = 2 * k + b
            wait_gather(b)
            nb = 1 - b

            @pl.when(i + 1 < nch)
            def _():
                wait_idx(nb)
                fire_gather(nb)

            for j in range(J):
                pltpu.async_copy(rows.at[b, pl.ds(j * 128, 128)],
                                 acc.at[dstb.at[b, j]], sem_s, add=True)
            for j in range(J):
                pltpu.make_async_copy(rows.at[b, pl.ds(j * 128, 128)],
                                      acc.at[dstb.at[b, j]], sem_s).wait()

            @pl.when(i + 2 < nch)
            def _():
                fire_idx(i + 2, b)

    plsc.subcore_barrier()
    pltpu.sync_copy(acc.at[pl.ds(r0, RPT)], out_hbm.at[cid, pl.ds(r0, RPT)])


@functools.partial(
    pl.kernel,
    out_type=jax.ShapeDtypeStruct((NCORE, NT), jnp.float32),
    mesh=_mesh,
    compiler_params=_sc_params,
    scratch_types=[
        pltpu.VMEM_SHARED((NT,), jnp.float32),
        pltpu.VMEM_SHARED((NT,), jnp.float32),
        pltpu.VMEM((RPT, 8), jnp.float32),
        pltpu.VMEM((RPT,), jnp.float32),
        pltpu.VMEM((2, CH), jnp.int32),
        pltpu.VMEM((2, J, 128), jnp.int32),
        pltpu.VMEM((2, CH), jnp.float32),
        pltpu.SemaphoreType.DMA,
        pltpu.SemaphoreType.DMA,
        pltpu.SemaphoreType.DMA,
    ],
)
def _pass_b(dst_hbm, src2_hbm, spart_hbm, zeros1_hbm, out_hbm,
            cacc, invt, spst, invb, dstb, srcb, vals, sem_i, sem_g, sem_s):
    cid = lax.axis_index("c")
    sid = lax.axis_index("s")
    r0 = sid * RPT
    pltpu.sync_copy(zeros1_hbm.at[pl.ds(r0, RPT)], cacc.at[pl.ds(r0, RPT)])

    six = jnp.full((16,), 6, jnp.int32)
    lane = lax.iota(jnp.int32, 16)
    pltpu.sync_copy(spart_hbm.at[0, pl.ds(r0, RPT)], spst)

    @pl.loop(0, RPT // 16)
    def _(k):
        o = k * 16
        invb[pl.ds(o, 16)] = plsc.load_gather(spst, [o + lane, six])

    pltpu.sync_copy(spart_hbm.at[1, pl.ds(r0, RPT)], spst)

    @pl.loop(0, RPT // 16)
    def _(k):
        o = k * 16
        g1 = plsc.load_gather(spst, [o + lane, six])
        invb[pl.ds(o, 16)] = 1.0 / (invb[pl.ds(o, 16)] + g1)

    pltpu.sync_copy(invb, invt.at[pl.ds(r0, RPT)])
    plsc.subcore_barrier()

    w = cid * NSUB + sid
    base = w * EPW
    nch = jnp.where(w == NWORK - 1, ITERS_LAST, ITERS)

    def fire_idx(i, b):
        off = base + i * CH
        pltpu.async_copy(dst_hbm.at[pl.ds(off, CH)], dstb.at[b], sem_i)
        row0 = pl.multiple_of(off // 128, 8)
        pltpu.async_copy(src2_hbm.at[pl.ds(row0, J)], srcb.at[b], sem_i)

    def wait_idx(b):
        pltpu.make_async_copy(dst_hbm.at[pl.ds(0, CH)], dstb.at[b],
                              sem_i).wait()
        pltpu.make_async_copy(src2_hbm.at[pl.ds(0, J)], srcb.at[b],
                              sem_i).wait()

    def fire_gather(b):
        pltpu.async_copy(invt.at[dstb.at[b]], vals.at[b], sem_g)

    def wait_gather(b):
        pltpu.make_async_copy(invt.at[dstb.at[b]], vals.at[b],
                              sem_g).wait()

    fire_idx(0, 0)
    wait_idx(0)
    fire_gather(0)
    fire_idx(1, 1)

    @pl.loop(0, nch // 2)
    def _(k):
        for b in range(2):
            i = 2 * k + b
            wait_gather(b)
            nb = 1 - b

            @pl.when(i + 1 < nch)
            def _():
                wait_idx(nb)
                fire_gather(nb)

            for j in range(J):
                pltpu.async_copy(vals.at[b, pl.ds(j * 128, 128)],
                                 cacc.at[srcb.at[b, j]], sem_s, add=True)
            for j in range(J):
                pltpu.make_async_copy(vals.at[b, pl.ds(j * 128, 128)],
                                      cacc.at[srcb.at[b, j]], sem_s).wait()

            @pl.when(i + 2 < nch)
            def _():
                fire_idx(i + 2, b)

    plsc.subcore_barrier()
    pltpu.sync_copy(cacc.at[pl.ds(r0, RPT)], out_hbm.at[cid, pl.ds(r0, RPT)])


BLK = 800


def _final_body(spart_ref, cpart_ref, w1_ref, b1_ref, w2_ref,
                b2_ref, out_ref, acc):
    i = pl.program_id(0)
    s = spart_ref[0] + spart_ref[1]
    inv = 1.0 / s[:, 6:7]
    c = cpart_ref[0, :, 0] + cpart_ref[1, :, 0]
    g = (c + inv[:, 0]) * (1.0 / N)
    rst = s * inv
    z = jnp.dot(rst, w1_ref[...], preferred_element_type=jnp.float32)
    h1 = jnp.maximum(z + b1_ref[...], 0.0)
    part = jnp.sum(h1 * g[:, None], axis=0, keepdims=True)

    @pl.when(i == 0)
    def _():
        acc[...] = jnp.zeros_like(acc)

    acc[...] += part

    @pl.when(i == pl.num_programs(0) - 1)
    def _():
        out_ref[...] = (
            jnp.dot(acc[...], w2_ref[...], preferred_element_type=jnp.float32)
            + b2_ref[...]
        )


def kernel(x, edge_index, W1, b1, W2, b2):
    src = edge_index[0].astype(jnp.int32)
    dst = edge_index[1].astype(jnp.int32)
    src2d = src.reshape(-1, 128)
    dst2d = dst.reshape(-1, 128)

    xpad = jnp.concatenate(
        [x.astype(jnp.float32),
         jnp.ones((N, 1), jnp.float32),
         jnp.zeros((N, 1), jnp.float32)], axis=1)
    xpad = jnp.pad(xpad, ((0, NPAD), (0, 0)))
    zeros8 = jnp.zeros((NT, 8), jnp.float32)
    zeros1 = jnp.zeros((NT,), jnp.float32)

    spart = _pass_a(src, dst2d, xpad, zeros8)
    cpart = _pass_b(dst, src2d, spart, zeros1)

    w1p = jnp.pad(W1.astype(jnp.float32), ((0, 2), (0, 0)))

    out = pl.pallas_call(
        _final_body,
        grid=(N // BLK,),
        in_specs=[
            pl.BlockSpec((NCORE, BLK, 8), lambda i: (0, i, 0)),
            pl.BlockSpec((NCORE, BLK, 1), lambda i: (0, i, 0)),
            pl.BlockSpec((8, HID), lambda i: (0, 0)),
            pl.BlockSpec((1, HID), lambda i: (0, 0)),
            pl.BlockSpec((HID, EMB), lambda i: (0, 0)),
            pl.BlockSpec((1, EMB), lambda i: (0, 0)),
        ],
        out_specs=pl.BlockSpec((1, EMB), lambda i: (0, 0)),
        out_shape=jax.ShapeDtypeStruct((1, EMB), jnp.float32),
        scratch_shapes=[pltpu.VMEM((1, EMB), jnp.float32)],
    )(spart, cpart[:, :, None], w1p, b1.reshape(1, HID),
      W2.astype(jnp.float32), b2.reshape(1, EMB))

    return out
